# Initial kernel scaffold; baseline (speedup 1.0000x reference)
import jax
import jax.numpy as jnp
from jax.experimental import pallas as pl


def kernel(features, edge_index, adj_values, weight):
    raise NotImplementedError("write your pallas kernel here")


# speedup vs baseline: 5.1215x; 5.1215x over previous
import functools

import jax
import jax.numpy as jnp
from jax import lax
from jax.experimental import pallas as pl
from jax.experimental.pallas import tpu as pltpu
from jax.experimental.pallas import tpu_sc as plsc

N = 10000
E = 320000
D = 128

NC = 2
NS = 16
NW = NC * NS
EP = E // NW
CH = 80
NCHUNK = EP // CH
RSTRIPE = N // NS


def _mm_body(x_ref, w_ref, o_ref):
    o_ref[...] = jnp.dot(x_ref[...], w_ref[...],
                         preferred_element_type=jnp.float32)


def _matmul(features, weight):
    bm = 2000
    return pl.pallas_call(
        _mm_body,
        grid=(N // bm,),
        in_specs=[
            pl.BlockSpec((bm, D), lambda i: (i, 0)),
            pl.BlockSpec((D, D), lambda i: (0, 0)),
        ],
        out_specs=pl.BlockSpec((bm, D), lambda i: (i, 0)),
        out_shape=jax.ShapeDtypeStruct((N, D), jnp.float32),
    )(features, weight)


def _comb_body(p_ref, o_ref):
    o_ref[...] = jnp.maximum(p_ref[0] + p_ref[1], 0.0)


def _combine_relu(partials):
    bm = 2000
    return pl.pallas_call(
        _comb_body,
        grid=(N // bm,),
        in_specs=[pl.BlockSpec((2, bm, D), lambda i: (0, i, 0))],
        out_specs=pl.BlockSpec((bm, D), lambda i: (i, 0)),
        out_shape=jax.ShapeDtypeStruct((N, D), jnp.float32),
    )(partials)


def _sc_body(support_hbm, row_hbm, col_hbm, adj_hbm, out_hbm,
             ridx_v, cidx_v, adj_v, rows_v, acc, sem):
    cid = lax.axis_index("c")
    sid = lax.axis_index("s")
    wid = cid * NS + sid
    ebase = wid * EP

    if True:
        zero16 = jnp.zeros((16,), jnp.float32)

        def zbody(i, _):
            for j in range(D // 16):
                rows_v[i, pl.ds(j * 16, 16)] = zero16
            return 0

        lax.fori_loop(0, CH, zbody, 0)
        nb = N // CH
        bper = (nb + NS - 1) // NS

        def zcopy(i, _):
            b = sid * bper + i

            @pl.when(b < nb)
            def _():
                pltpu.sync_copy(rows_v, acc.at[pl.ds(b * CH, CH)])

            return 0

        lax.fori_loop(0, bper, zcopy, 0)
        plsc.subcore_barrier()

        pltpu.sync_copy(adj_hbm.at[pl.ds(ebase, EP)], adj_v)

        def chunk_body(c, _):
            gbase = ebase + c * CH
            pltpu.sync_copy(row_hbm.at[pl.ds(gbase, CH)], ridx_v)
            pltpu.sync_copy(col_hbm.at[pl.ds(gbase, CH)], cidx_v)
            pltpu.async_copy(support_hbm.at[cidx_v], rows_v, sem).wait()

            def scale_body(kb, _):
                av = adj_v[pl.ds(c * CH + kb * 16, 16)]
                dnums = lax.GatherDimensionNumbers(
                    offset_dims=(), collapsed_slice_dims=(0,),
                    start_index_map=(0,))
                for i in range(16):
                    a = lax.gather(
                        av, jnp.full((16, 1), i, jnp.int32), dnums,
                        slice_sizes=(1,),
                        mode=lax.GatherScatterMode.PROMISE_IN_BOUNDS)
                    k = kb * 16 + i
                    for j in range(D // 16):
                        rows_v[k, pl.ds(j * 16, 16)] = (
                            rows_v[k, pl.ds(j * 16, 16)] * a)
                return 0

            lax.fori_loop(0, CH // 16, scale_body, 0)
            pltpu.sync_copy(rows_v, acc.at[ridx_v], add=True)
            return 0

        lax.fori_loop(0, NCHUNK, chunk_body, 0)
        plsc.subcore_barrier()

        def wcopy(i, _):
            b = sid * bper + i

            @pl.when(b < nb)
            def _():
                pltpu.sync_copy(acc.at[pl.ds(b * CH, CH)],
                                out_hbm.at[cid, pl.ds(b * CH, CH)])

            return 0

        lax.fori_loop(0, bper, wcopy, 0)


@functools.partial(
    pl.kernel,
    out_type=jax.ShapeDtypeStruct((NC, N, D), jnp.float32),
    mesh=plsc.VectorSubcoreMesh(core_axis_name="c", subcore_axis_name="s"),
    scratch_types=[
        pltpu.VMEM((CH,), jnp.int32),
        pltpu.VMEM((CH,), jnp.int32),
        pltpu.VMEM((EP,), jnp.float32),
        pltpu.VMEM((CH, D), jnp.float32),
        pltpu.VMEM_SHARED((N, D), jnp.float32),
        pltpu.SemaphoreType.DMA,
    ],
)
def _sc_aggregate(support_hbm, row_hbm, col_hbm, adj_hbm, out_hbm,
                  ridx_v, cidx_v, adj_v, rows_v, acc, sem):
    _sc_body(support_hbm, row_hbm, col_hbm, adj_hbm, out_hbm,
             ridx_v, cidx_v, adj_v, rows_v, acc, sem)


def kernel(features, edge_index, adj_values, weight):
    support = _matmul(features, weight)
    row = edge_index[0]
    col = edge_index[1]
    partials = _sc_aggregate(support, row, col, adj_values)
    return _combine_relu(partials)


# speedup vs baseline: 7.7052x; 1.5045x over previous
import functools

import jax
import jax.numpy as jnp
from jax import lax
from jax.experimental import pallas as pl
from jax.experimental.pallas import tpu as pltpu
from jax.experimental.pallas import tpu_sc as plsc

N = 10000
E = 320000
D = 128

NC = 2
NS = 16
NW = NC * NS
EP = E // NW
CH = 80
NCHUNK = EP // CH


def _out_body(p_ref, w_ref, o_ref):
    agg = p_ref[0] + p_ref[1]
    o_ref[...] = jnp.maximum(
        jnp.dot(agg, w_ref[...], preferred_element_type=jnp.float32), 0.0)


def _fused_out(partials, weight):
    bm = 2000
    return pl.pallas_call(
        _out_body,
        grid=(N // bm,),
        in_specs=[
            pl.BlockSpec((2, bm, D), lambda i: (0, i, 0)),
            pl.BlockSpec((D, D), lambda i: (0, 0)),
        ],
        out_specs=pl.BlockSpec((bm, D), lambda i: (i, 0)),
        out_shape=jax.ShapeDtypeStruct((N, D), jnp.float32),
    )(partials, weight)


def _sc_body(feat_hbm, row_hbm, col_hbm, adj_hbm, out_hbm,
             ridx_a, cidx_a, ridx_b, cidx_b, adj_v, rows_a, rows_b, acc,
             sem_ga, sem_gb):
    cid = lax.axis_index("c")
    sid = lax.axis_index("s")
    wid = cid * NS + sid
    ebase = wid * EP

    zero16 = jnp.zeros((16,), jnp.float32)

    def zbody(i, _):
        for j in range(D // 16):
            rows_a[i, pl.ds(j * 16, 16)] = zero16
        return 0

    lax.fori_loop(0, CH, zbody, 0)
    nb = N // CH
    bper = (nb + NS - 1) // NS

    def zcopy(i, _):
        b = sid * bper + i

        @pl.when(b < nb)
        def _():
            pltpu.sync_copy(rows_a, acc.at[pl.ds(b * CH, CH)])

        return 0

    lax.fori_loop(0, bper, zcopy, 0)

    pltpu.sync_copy(adj_hbm.at[pl.ds(ebase, EP)], adj_v)

    def stage_idx(c, ridx, cidx):
        pltpu.sync_copy(row_hbm.at[pl.ds(ebase + c * CH, CH)], ridx)
        pltpu.sync_copy(col_hbm.at[pl.ds(ebase + c * CH, CH)], cidx)

    def issue_gather(cidx, buf, sem):
        pltpu.async_copy(feat_hbm.at[cidx], buf, sem)

    def wait_gather(buf, sem):
        pltpu.make_async_copy(feat_hbm.at[cidx_a], buf, sem).wait()

    _DNUMS = lax.GatherDimensionNumbers(
        offset_dims=(), collapsed_slice_dims=(0,), start_index_map=(0,))

    def scale(buf, c):
        def scale_body(kb, _):
            av = adj_v[pl.ds(c * CH + kb * 16, 16)]
            for i in range(16):
                a = lax.gather(
                    av, jnp.full((16, 1), i, jnp.int32), _DNUMS,
                    slice_sizes=(1,),
                    mode=lax.GatherScatterMode.PROMISE_IN_BOUNDS)
                k = kb * 16 + i
                for j in range(D // 16):
                    buf[k, pl.ds(j * 16, 16)] = (
                        buf[k, pl.ds(j * 16, 16)] * a)
            return 0

        lax.fori_loop(0, CH // 16, scale_body, 0)

    def scatter(buf, ridx):
        pltpu.sync_copy(buf, acc.at[ridx], add=True)

    stage_idx(0, ridx_a, cidx_a)
    stage_idx(1, ridx_b, cidx_b)
    issue_gather(cidx_a, rows_a, sem_ga)
    issue_gather(cidx_b, rows_b, sem_gb)
    plsc.subcore_barrier()

    npair = (NCHUNK - 1) // 2

    def pair_body(i, _):
        c0 = 2 * i
        c1 = c0 + 1
        wait_gather(rows_a, sem_ga)
        scale(rows_a, c0)
        scatter(rows_a, ridx_a)
        stage_idx(c0 + 2, ridx_a, cidx_a)
        issue_gather(cidx_a, rows_a, sem_ga)
        wait_gather(rows_b, sem_gb)
        scale(rows_b, c1)
        scatter(rows_b, ridx_b)

        @pl.when(c1 + 2 < NCHUNK)
        def _():
            stage_idx(c1 + 2, ridx_b, cidx_b)
            issue_gather(cidx_b, rows_b, sem_gb)

        return 0

    lax.fori_loop(0, npair, pair_body, 0)
    wait_gather(rows_a, sem_ga)
    scale(rows_a, NCHUNK - 1)
    scatter(rows_a, ridx_a)
    plsc.subcore_barrier()

    def wcopy(i, _):
        b = sid * bper + i

        @pl.when(b < nb)
        def _():
            pltpu.sync_copy(acc.at[pl.ds(b * CH, CH)],
                            out_hbm.at[cid, pl.ds(b * CH, CH)])

        return 0

    lax.fori_loop(0, bper, wcopy, 0)


@functools.partial(
    pl.kernel,
    out_type=jax.ShapeDtypeStruct((NC, N, D), jnp.float32),
    mesh=plsc.VectorSubcoreMesh(core_axis_name="c", subcore_axis_name="s"),
    scratch_types=[
        pltpu.VMEM((CH,), jnp.int32),
        pltpu.VMEM((CH,), jnp.int32),
        pltpu.VMEM((CH,), jnp.int32),
        pltpu.VMEM((CH,), jnp.int32),
        pltpu.VMEM((EP,), jnp.float32),
        pltpu.VMEM((CH, D), jnp.float32),
        pltpu.VMEM((CH, D), jnp.float32),
        pltpu.VMEM_SHARED((N, D), jnp.float32),
        pltpu.SemaphoreType.DMA,
        pltpu.SemaphoreType.DMA,
    ],
)
def _sc_aggregate(feat_hbm, row_hbm, col_hbm, adj_hbm, out_hbm,
                  ridx_a, cidx_a, ridx_b, cidx_b, adj_v, rows_a, rows_b,
                  acc, sem_ga, sem_gb):
    _sc_body(feat_hbm, row_hbm, col_hbm, adj_hbm, out_hbm,
             ridx_a, cidx_a, ridx_b, cidx_b, adj_v, rows_a, rows_b, acc,
             sem_ga, sem_gb)


def kernel(features, edge_index, adj_values, weight):
    row = edge_index[0]
    col = edge_index[1]
    partials = _sc_aggregate(features, row, col, adj_values)
    return _fused_out(partials, weight)


# speedup vs baseline: 9.0391x; 1.1731x over previous
import functools

import jax
import jax.numpy as jnp
from jax import lax
from jax.experimental import pallas as pl
from jax.experimental.pallas import tpu as pltpu
from jax.experimental.pallas import tpu_sc as plsc

N = 10000
E = 320000
D = 128

NC = 2
NS = 16
NW = NC * NS
EP = E // NW
CH = 80
NCHUNK = EP // CH


def _out_body(p_ref, w_ref, o_ref):
    agg = p_ref[0] + p_ref[1]
    o_ref[...] = jnp.maximum(
        jnp.dot(agg, w_ref[...], preferred_element_type=jnp.float32), 0.0)


def _fused_out(partials, weight):
    bm = 2000
    return pl.pallas_call(
        _out_body,
        grid=(N // bm,),
        in_specs=[
            pl.BlockSpec((2, bm, D), lambda i: (0, i, 0)),
            pl.BlockSpec((D, D), lambda i: (0, 0)),
        ],
        out_specs=pl.BlockSpec((bm, D), lambda i: (i, 0)),
        out_shape=jax.ShapeDtypeStruct((N, D), jnp.float32),
    )(partials, weight)


def _sc_body(feat_hbm, row_hbm, col_hbm, adj_hbm, out_hbm,
             ridx_a, cidx_a, ridx_b, cidx_b, adj_v, rows_a, rows_b, acc,
             sem_ga, sem_gb, sem_ia, sem_ib):
    cid = lax.axis_index("c")
    sid = lax.axis_index("s")
    wid = cid * NS + sid
    ebase = wid * EP

    zero16 = jnp.zeros((16,), jnp.float32)

    def zbody(i, _):
        for j in range(D // 16):
            rows_a[i, pl.ds(j * 16, 16)] = zero16
        return 0

    lax.fori_loop(0, CH, zbody, 0)
    nb = N // CH
    bper = (nb + NS - 1) // NS

    def zcopy(i, _):
        b = sid * bper + i

        @pl.when(b < nb)
        def _():
            pltpu.sync_copy(rows_a, acc.at[pl.ds(b * CH, CH)])

        return 0

    lax.fori_loop(0, bper, zcopy, 0)

    pltpu.sync_copy(adj_hbm.at[pl.ds(ebase, EP)], adj_v)

    def stage_idx(c, ridx, cidx):
        pltpu.sync_copy(row_hbm.at[pl.ds(ebase + c * CH, CH)], ridx)
        pltpu.sync_copy(col_hbm.at[pl.ds(ebase + c * CH, CH)], cidx)

    def stage_idx_async(c, ridx, cidx, sem):
        pltpu.async_copy(row_hbm.at[pl.ds(ebase + c * CH, CH)], ridx, sem)
        pltpu.async_copy(col_hbm.at[pl.ds(ebase + c * CH, CH)], cidx, sem)

    def wait_idx(ridx, cidx, sem):
        pltpu.make_async_copy(row_hbm.at[pl.ds(ebase, CH)], ridx, sem).wait()
        pltpu.make_async_copy(col_hbm.at[pl.ds(ebase, CH)], cidx, sem).wait()

    def issue_gather(cidx, buf, sem):
        pltpu.async_copy(feat_hbm.at[cidx], buf, sem)

    def wait_gather(buf, sem):
        pltpu.make_async_copy(feat_hbm.at[cidx_a], buf, sem).wait()

    _DNUMS = lax.GatherDimensionNumbers(
        offset_dims=(), collapsed_slice_dims=(0,), start_index_map=(0,))

    def scale(buf, c):
        def scale_body(kb, _):
            av = adj_v[pl.ds(c * CH + kb * 16, 16)]
            for i in range(16):
                a = lax.gather(
                    av, jnp.full((16, 1), i, jnp.int32), _DNUMS,
                    slice_sizes=(1,),
                    mode=lax.GatherScatterMode.PROMISE_IN_BOUNDS)
                k = kb * 16 + i
                for j in range(D // 16):
                    buf[k, pl.ds(j * 16, 16)] = (
                        buf[k, pl.ds(j * 16, 16)] * a)
            return 0

        lax.fori_loop(0, CH // 16, scale_body, 0)

    def scatter(buf, ridx):
        pltpu.sync_copy(buf, acc.at[ridx], add=True)

    stage_idx(0, ridx_a, cidx_a)
    stage_idx(1, ridx_b, cidx_b)
    issue_gather(cidx_a, rows_a, sem_ga)
    issue_gather(cidx_b, rows_b, sem_gb)
    plsc.subcore_barrier()

    npair = (NCHUNK - 1) // 2

    def pair_body(i, _):
        c0 = 2 * i
        c1 = c0 + 1
        wait_gather(rows_a, sem_ga)
        scale(rows_a, c0)
        scatter(rows_a, ridx_a)
        stage_idx_async(c0 + 2, ridx_a, cidx_a, sem_ia)
        wait_gather(rows_b, sem_gb)
        scale(rows_b, c1)
        scatter(rows_b, ridx_b)

        @pl.when(c1 + 2 < NCHUNK)
        def _():
            stage_idx_async(c1 + 2, ridx_b, cidx_b, sem_ib)

        wait_idx(ridx_a, cidx_a, sem_ia)
        issue_gather(cidx_a, rows_a, sem_ga)

        @pl.when(c1 + 2 < NCHUNK)
        def _():
            wait_idx(ridx_b, cidx_b, sem_ib)
            issue_gather(cidx_b, rows_b, sem_gb)

        return 0

    lax.fori_loop(0, npair, pair_body, 0)
    wait_gather(rows_a, sem_ga)
    scale(rows_a, NCHUNK - 1)
    scatter(rows_a, ridx_a)
    plsc.subcore_barrier()

    def wcopy(i, _):
        b = sid * bper + i

        @pl.when(b < nb)
        def _():
            pltpu.sync_copy(acc.at[pl.ds(b * CH, CH)],
                            out_hbm.at[cid, pl.ds(b * CH, CH)])

        return 0

    lax.fori_loop(0, bper, wcopy, 0)


@functools.partial(
    pl.kernel,
    out_type=jax.ShapeDtypeStruct((NC, N, D), jnp.float32),
    mesh=plsc.VectorSubcoreMesh(core_axis_name="c", subcore_axis_name="s"),
    scratch_types=[
        pltpu.VMEM((CH,), jnp.int32),
        pltpu.VMEM((CH,), jnp.int32),
        pltpu.VMEM((CH,), jnp.int32),
        pltpu.VMEM((CH,), jnp.int32),
        pltpu.VMEM((EP,), jnp.float32),
        pltpu.VMEM((CH, D), jnp.float32),
        pltpu.VMEM((CH, D), jnp.float32),
        pltpu.VMEM_SHARED((N, D), jnp.float32),
        pltpu.SemaphoreType.DMA,
        pltpu.SemaphoreType.DMA,
        pltpu.SemaphoreType.DMA,
        pltpu.SemaphoreType.DMA,
    ],
)
def _sc_aggregate(feat_hbm, row_hbm, col_hbm, adj_hbm, out_hbm,
                  ridx_a, cidx_a, ridx_b, cidx_b, adj_v, rows_a, rows_b,
                  acc, sem_ga, sem_gb, sem_ia, sem_ib):
    _sc_body(feat_hbm, row_hbm, col_hbm, adj_hbm, out_hbm,
             ridx_a, cidx_a, ridx_b, cidx_b, adj_v, rows_a, rows_b, acc,
             sem_ga, sem_gb, sem_ia, sem_ib)


def kernel(features, edge_index, adj_values, weight):
    row = edge_index[0]
    col = edge_index[1]
    partials = _sc_aggregate(features, row, col, adj_values)
    return _fused_out(partials, weight)


# speedup vs baseline: 9.3885x; 1.0386x over previous
import functools

import jax
import jax.numpy as jnp
from jax import lax
from jax.experimental import pallas as pl
from jax.experimental.pallas import tpu as pltpu
from jax.experimental.pallas import tpu_sc as plsc

N = 10000
E = 320000
D = 128

NC = 2
NS = 16
NW = NC * NS
EP = E // NW
CH = 128
NFULL = EP // CH
TAIL = EP - NFULL * CH
ZB = 80


def _out_body(p_ref, w_ref, o_ref):
    agg = p_ref[0] + p_ref[1]
    o_ref[...] = jnp.maximum(
        jnp.dot(agg, w_ref[...], preferred_element_type=jnp.float32), 0.0)


def _fused_out(partials, weight):
    bm = 2000
    return pl.pallas_call(
        _out_body,
        grid=(N // bm,),
        in_specs=[
            pl.BlockSpec((2, bm, D), lambda i: (0, i, 0)),
            pl.BlockSpec((D, D), lambda i: (0, 0)),
        ],
        out_specs=pl.BlockSpec((bm, D), lambda i: (i, 0)),
        out_shape=jax.ShapeDtypeStruct((N, D), jnp.float32),
    )(partials, weight)


def _sc_body(feat_hbm, row_hbm, col_hbm, adj_hbm, out_hbm,
             ridx_a, cidx_a, ridx_b, cidx_b, adj_a, adj_b,
             ridx_t, cidx_t, adj_t, rows_a, rows_b, acc,
             sem_ga, sem_gb, sem_ia, sem_ib):
    cid = lax.axis_index("c")
    sid = lax.axis_index("s")
    wid = cid * NS + sid
    ebase = wid * EP

    zero16 = jnp.zeros((16,), jnp.float32)

    def zbody(i, _):
        for j in range(D // 16):
            rows_a[i, pl.ds(j * 16, 16)] = zero16
        return 0

    lax.fori_loop(0, CH, zbody, 0)
    nb = N // ZB
    bper = (nb + NS - 1) // NS

    def zcopy(i, _):
        b = sid * bper + i

        @pl.when(b < nb)
        def _():
            pltpu.sync_copy(rows_a.at[pl.ds(0, ZB)],
                            acc.at[pl.ds(b * ZB, ZB)])

        return 0

    lax.fori_loop(0, bper, zcopy, 0)

    def stage_idx(c, ridx, cidx, adj):
        pltpu.sync_copy(row_hbm.at[pl.ds(ebase + c * CH, CH)], ridx)
        pltpu.sync_copy(col_hbm.at[pl.ds(ebase + c * CH, CH)], cidx)
        pltpu.sync_copy(adj_hbm.at[pl.ds(ebase + c * CH, CH)], adj)

    def stage_idx_async(c, ridx, cidx, adj, sem):
        pltpu.async_copy(row_hbm.at[pl.ds(ebase + c * CH, CH)], ridx, sem)
        pltpu.async_copy(col_hbm.at[pl.ds(ebase + c * CH, CH)], cidx, sem)
        pltpu.async_copy(adj_hbm.at[pl.ds(ebase + c * CH, CH)], adj, sem)

    def wait_idx(ridx, cidx, adj, sem):
        pltpu.make_async_copy(row_hbm.at[pl.ds(ebase, CH)], ridx, sem).wait()
        pltpu.make_async_copy(col_hbm.at[pl.ds(ebase, CH)], cidx, sem).wait()
        pltpu.make_async_copy(adj_hbm.at[pl.ds(ebase, CH)], adj, sem).wait()

    def issue_gather(cidx, buf, sem):
        pltpu.async_copy(feat_hbm.at[cidx], buf, sem)

    def wait_gather(buf, sem):
        pltpu.make_async_copy(feat_hbm.at[cidx_a], buf, sem).wait()

    _DNUMS = lax.GatherDimensionNumbers(
        offset_dims=(), collapsed_slice_dims=(0,), start_index_map=(0,))

    def scale_block(buf, adj, kb):
        av = adj[pl.ds(kb * 16, 16)]
        for i in range(16):
            a = lax.gather(
                av, jnp.full((16, 1), i, jnp.int32), _DNUMS,
                slice_sizes=(1,),
                mode=lax.GatherScatterMode.PROMISE_IN_BOUNDS)
            k = kb * 16 + i
            for j in range(D // 16):
                buf[k, pl.ds(j * 16, 16)] = (
                    buf[k, pl.ds(j * 16, 16)] * a)

    def scale(buf, adj):
        def scale_body(kb, _):
            scale_block(buf, adj, kb)
            return 0

        lax.fori_loop(0, CH // 16, scale_body, 0)

    def scatter(buf, ridx):
        pltpu.sync_copy(buf, acc.at[ridx], add=True)

    stage_idx(0, ridx_a, cidx_a, adj_a)
    stage_idx(1, ridx_b, cidx_b, adj_b)
    issue_gather(cidx_a, rows_a, sem_ga)
    issue_gather(cidx_b, rows_b, sem_gb)
    plsc.subcore_barrier()

    npair = NFULL // 2

    def pair_body(i, _):
        c0 = 2 * i
        c1 = c0 + 1
        wait_gather(rows_a, sem_ga)
        scale(rows_a, adj_a)
        scatter(rows_a, ridx_a)

        @pl.when(c0 + 2 < NFULL)
        def _():
            stage_idx_async(c0 + 2, ridx_a, cidx_a, adj_a, sem_ia)

        wait_gather(rows_b, sem_gb)
        scale(rows_b, adj_b)
        scatter(rows_b, ridx_b)

        @pl.when(c1 + 2 < NFULL)
        def _():
            stage_idx_async(c1 + 2, ridx_b, cidx_b, adj_b, sem_ib)

        @pl.when(c0 + 2 < NFULL)
        def _():
            wait_idx(ridx_a, cidx_a, adj_a, sem_ia)
            issue_gather(cidx_a, rows_a, sem_ga)

        @pl.when(c1 + 2 < NFULL)
        def _():
            wait_idx(ridx_b, cidx_b, adj_b, sem_ib)
            issue_gather(cidx_b, rows_b, sem_gb)

        return 0

    lax.fori_loop(0, npair, pair_body, 0)

    stage_idx_tail = ebase + NFULL * CH
    pltpu.sync_copy(row_hbm.at[pl.ds(stage_idx_tail, TAIL)], ridx_t)
    pltpu.sync_copy(col_hbm.at[pl.ds(stage_idx_tail, TAIL)], cidx_t)
    pltpu.sync_copy(adj_hbm.at[pl.ds(stage_idx_tail, TAIL)], adj_t)
    pltpu.async_copy(feat_hbm.at[cidx_t],
                     rows_a.at[pl.ds(0, TAIL)], sem_ga).wait()
    for kb in range(TAIL // 16):
        scale_block(rows_a, adj_t, kb)
    pltpu.sync_copy(rows_a.at[pl.ds(0, TAIL)], acc.at[ridx_t], add=True)
    plsc.subcore_barrier()

    def wcopy(i, _):
        b = sid * bper + i

        @pl.when(b < nb)
        def _():
            pltpu.sync_copy(acc.at[pl.ds(b * ZB, ZB)],
                            out_hbm.at[cid, pl.ds(b * ZB, ZB)])

        return 0

    lax.fori_loop(0, bper, wcopy, 0)


@functools.partial(
    pl.kernel,
    out_type=jax.ShapeDtypeStruct((NC, N, D), jnp.float32),
    mesh=plsc.VectorSubcoreMesh(core_axis_name="c", subcore_axis_name="s"),
    scratch_types=[
        pltpu.VMEM((CH,), jnp.int32),
        pltpu.VMEM((CH,), jnp.int32),
        pltpu.VMEM((CH,), jnp.int32),
        pltpu.VMEM((CH,), jnp.int32),
        pltpu.VMEM((CH,), jnp.float32),
        pltpu.VMEM((CH,), jnp.float32),
        pltpu.VMEM((TAIL,), jnp.int32),
        pltpu.VMEM((TAIL,), jnp.int32),
        pltpu.VMEM((TAIL,), jnp.float32),
        pltpu.VMEM((CH, D), jnp.float32),
        pltpu.VMEM((CH, D), jnp.float32),
        pltpu.VMEM_SHARED((N, D), jnp.float32),
        pltpu.SemaphoreType.DMA,
        pltpu.SemaphoreType.DMA,
        pltpu.SemaphoreType.DMA,
        pltpu.SemaphoreType.DMA,
    ],
)
def _sc_aggregate(feat_hbm, row_hbm, col_hbm, adj_hbm, out_hbm,
                  ridx_a, cidx_a, ridx_b, cidx_b, adj_a, adj_b,
                  ridx_t, cidx_t, adj_t, rows_a, rows_b,
                  acc, sem_ga, sem_gb, sem_ia, sem_ib):
    _sc_body(feat_hbm, row_hbm, col_hbm, adj_hbm, out_hbm,
             ridx_a, cidx_a, ridx_b, cidx_b, adj_a, adj_b,
             ridx_t, cidx_t, adj_t, rows_a, rows_b, acc,
             sem_ga, sem_gb, sem_ia, sem_ib)


def kernel(features, edge_index, adj_values, weight):
    row = edge_index[0]
    col = edge_index[1]
    partials = _sc_aggregate(features, row, col, adj_values)
    return _fused_out(partials, weight)


# speedup vs baseline: 12.1675x; 1.2960x over previous
import functools

import jax
import jax.numpy as jnp
from jax import lax
from jax.experimental import pallas as pl
from jax.experimental.pallas import tpu as pltpu
from jax.experimental.pallas import tpu_sc as plsc

N = 10000
E = 320000
D = 128

NC = 2
NS = 16
NW = NC * NS
EP = E // NW
CH = 128
NFULL = EP // CH
TAIL = EP - NFULL * CH
ZB = 80


def _out_body(p_ref, w_ref, o_ref):
    agg = p_ref[0] + p_ref[1]
    o_ref[...] = jnp.maximum(
        jnp.dot(agg, w_ref[...], preferred_element_type=jnp.float32), 0.0)


def _fused_out(partials, weight):
    bm = 2000
    return pl.pallas_call(
        _out_body,
        grid=(N // bm,),
        in_specs=[
            pl.BlockSpec((2, bm, D), lambda i: (0, i, 0)),
            pl.BlockSpec((D, D), lambda i: (0, 0)),
        ],
        out_specs=pl.BlockSpec((bm, D), lambda i: (i, 0)),
        out_shape=jax.ShapeDtypeStruct((N, D), jnp.float32),
    )(partials, weight)


def _sc_body(feat_hbm, row_hbm, col_hbm, adj_hbm, out_hbm,
             ridx_a, cidx_a, ridx_b, cidx_b, adj_a, adj_b,
             ridx_t, cidx_t, adj_t, rows_a, rows_b, acc,
             sem_ga, sem_gb, sem_ia, sem_ib):
    cid = lax.axis_index("c")
    sid = lax.axis_index("s")
    wid = cid * NS + sid
    ebase = wid * EP

    zero16 = jnp.zeros((16,), jnp.float32)

    def zbody(i, _):
        for j in range(D // 16):
            rows_a[i, pl.ds(j * 16, 16)] = zero16
        return 0

    lax.fori_loop(0, CH, zbody, 0)
    nb = N // ZB
    bper = (nb + NS - 1) // NS

    def zcopy(i, _):
        b = sid * bper + i

        @pl.when(b < nb)
        def _():
            pltpu.sync_copy(rows_a.at[pl.ds(0, ZB)],
                            acc.at[pl.ds(b * ZB, ZB)])

        return 0

    lax.fori_loop(0, bper, zcopy, 0)

    def stage_cidx(c, cidx):
        pltpu.sync_copy(col_hbm.at[pl.ds(ebase + c * CH, CH)], cidx)

    def stage_cidx_async(c, cidx, sem):
        pltpu.async_copy(col_hbm.at[pl.ds(ebase + c * CH, CH)], cidx, sem)

    def wait_cidx(cidx, sem):
        pltpu.make_async_copy(col_hbm.at[pl.ds(ebase, CH)], cidx, sem).wait()

    def stage_ra_async(c, ridx, adj, sem):
        pltpu.async_copy(row_hbm.at[pl.ds(ebase + c * CH, CH)], ridx, sem)
        pltpu.async_copy(adj_hbm.at[pl.ds(ebase + c * CH, CH)], adj, sem)

    def wait_ra(ridx, adj, sem):
        pltpu.make_async_copy(row_hbm.at[pl.ds(ebase, CH)], ridx, sem).wait()
        pltpu.make_async_copy(adj_hbm.at[pl.ds(ebase, CH)], adj, sem).wait()

    def issue_gather(cidx, buf, sem):
        pltpu.async_copy(feat_hbm.at[cidx], buf, sem)

    def wait_gather(buf, sem):
        pltpu.make_async_copy(feat_hbm.at[cidx_a], buf, sem).wait()

    _DNUMS = lax.GatherDimensionNumbers(
        offset_dims=(), collapsed_slice_dims=(0,), start_index_map=(0,))

    def scale_block(buf, adj, kb):
        av = adj[pl.ds(kb * 16, 16)]
        for i in range(16):
            a = lax.gather(
                av, jnp.full((16, 1), i, jnp.int32), _DNUMS,
                slice_sizes=(1,),
                mode=lax.GatherScatterMode.PROMISE_IN_BOUNDS)
            k = kb * 16 + i
            for j in range(D // 16):
                buf[k, pl.ds(j * 16, 16)] = (
                    buf[k, pl.ds(j * 16, 16)] * a)

    def scale(buf, adj):
        def scale_body(kb, _):
            scale_block(buf, adj, kb)
            return 0

        lax.fori_loop(0, CH // 16, scale_body, 0)

    def scatter(buf, ridx):
        pltpu.sync_copy(buf, acc.at[ridx], add=True)

    stage_cidx(0, cidx_a)
    stage_cidx(1, cidx_b)
    issue_gather(cidx_a, rows_a, sem_ga)
    issue_gather(cidx_b, rows_b, sem_gb)
    stage_ra_async(0, ridx_a, adj_a, sem_ia)
    stage_ra_async(1, ridx_b, adj_b, sem_ib)
    plsc.subcore_barrier()

    npair = NFULL // 2

    def half(c, rows, ridx, cidx, adj, sem_g, sem_i):
        wait_gather(rows, sem_g)
        wait_ra(ridx, adj, sem_i)

        @pl.when(c + 2 < NFULL)
        def _():
            stage_cidx_async(c + 2, cidx, sem_i)

        scale(rows, adj)
        scatter(rows, ridx)

        @pl.when(c + 2 < NFULL)
        def _():
            wait_cidx(cidx, sem_i)
            issue_gather(cidx, rows, sem_g)
            stage_ra_async(c + 2, ridx, adj, sem_i)

    def pair_body(i, _):
        c0 = 2 * i
        half(c0, rows_a, ridx_a, cidx_a, adj_a, sem_ga, sem_ia)
        half(c0 + 1, rows_b, ridx_b, cidx_b, adj_b, sem_gb, sem_ib)
        return 0

    lax.fori_loop(0, npair, pair_body, 0)

    stage_idx_tail = ebase + NFULL * CH
    pltpu.sync_copy(row_hbm.at[pl.ds(stage_idx_tail, TAIL)], ridx_t)
    pltpu.sync_copy(col_hbm.at[pl.ds(stage_idx_tail, TAIL)], cidx_t)
    pltpu.sync_copy(adj_hbm.at[pl.ds(stage_idx_tail, TAIL)], adj_t)
    pltpu.async_copy(feat_hbm.at[cidx_t],
                     rows_a.at[pl.ds(0, TAIL)], sem_ga).wait()
    for kb in range(TAIL // 16):
        scale_block(rows_a, adj_t, kb)
    pltpu.sync_copy(rows_a.at[pl.ds(0, TAIL)], acc.at[ridx_t], add=True)
    plsc.subcore_barrier()

    def wcopy(i, _):
        b = sid * bper + i

        @pl.when(b < nb)
        def _():
            pltpu.sync_copy(acc.at[pl.ds(b * ZB, ZB)],
                            out_hbm.at[cid, pl.ds(b * ZB, ZB)])

        return 0

    lax.fori_loop(0, bper, wcopy, 0)


@functools.partial(
    pl.kernel,
    out_type=jax.ShapeDtypeStruct((NC, N, D), jnp.float32),
    mesh=plsc.VectorSubcoreMesh(core_axis_name="c", subcore_axis_name="s"),
    scratch_types=[
        pltpu.VMEM((CH,), jnp.int32),
        pltpu.VMEM((CH,), jnp.int32),
        pltpu.VMEM((CH,), jnp.int32),
        pltpu.VMEM((CH,), jnp.int32),
        pltpu.VMEM((CH,), jnp.float32),
        pltpu.VMEM((CH,), jnp.float32),
        pltpu.VMEM((TAIL,), jnp.int32),
        pltpu.VMEM((TAIL,), jnp.int32),
        pltpu.VMEM((TAIL,), jnp.float32),
        pltpu.VMEM((CH, D), jnp.float32),
        pltpu.VMEM((CH, D), jnp.float32),
        pltpu.VMEM_SHARED((N, D), jnp.float32),
        pltpu.SemaphoreType.DMA,
        pltpu.SemaphoreType.DMA,
        pltpu.SemaphoreType.DMA,
        pltpu.SemaphoreType.DMA,
    ],
)
def _sc_aggregate(feat_hbm, row_hbm, col_hbm, adj_hbm, out_hbm,
                  ridx_a, cidx_a, ridx_b, cidx_b, adj_a, adj_b,
                  ridx_t, cidx_t, adj_t, rows_a, rows_b,
                  acc, sem_ga, sem_gb, sem_ia, sem_ib):
    _sc_body(feat_hbm, row_hbm, col_hbm, adj_hbm, out_hbm,
             ridx_a, cidx_a, ridx_b, cidx_b, adj_a, adj_b,
             ridx_t, cidx_t, adj_t, rows_a, rows_b, acc,
             sem_ga, sem_gb, sem_ia, sem_ib)


def kernel(features, edge_index, adj_values, weight):
    row = edge_index[0]
    col = edge_index[1]
    partials = _sc_aggregate(features, row, col, adj_values)
    return _fused_out(partials, weight)
